# baseline (speedup 1.0000x reference)
import jax
import jax.numpy as jnp
from jax.experimental import pallas as pl


def _bias_add_body(a_ref, b_ref, o_ref):
    o_ref[...] = a_ref[...] + b_ref[...]


def kernel(x_b, table, hash_values, input_order, bias):
    batch_size = x_b.shape[0]
    classes, fpd, entries = table.shape
    hfn = hash_values.shape[0]
    fin = hash_values.shape[1]
    mapped = x_b[:, input_order]
    hash_inputs = mapped.reshape(batch_size * fpd, fin)
    masked = hash_inputs[:, None, :] * hash_values[None, :, :]
    out = jnp.zeros(masked.shape[:2], dtype=jnp.int32)
    for i in range(masked.shape[2]):
        out = jnp.bitwise_xor(out, masked[:, :, i])
    filter_idx = out.reshape(batch_size, fpd, hfn).transpose(1, 0, 2).reshape(1, fpd, batch_size * hfn)
    filter_idx = jnp.broadcast_to(filter_idx, (classes, fpd, batch_size * hfn))
    flat_lookup = jnp.take_along_axis(table, filter_idx, axis=2)
    lookup = flat_lookup.reshape(classes, fpd, batch_size, hfn).transpose(2, 0, 1, 3)
    bin_lookup = (lookup >= 0).astype(jnp.float32) * 2.0 - 1.0
    reduced = jnp.min(bin_lookup, axis=-1)
    acts = reduced.sum(axis=2)
    bias_b = jnp.broadcast_to(bias[None, :], acts.shape)
    return pl.pallas_call(
        _bias_add_body,
        out_shape=jax.ShapeDtypeStruct(acts.shape, acts.dtype),
    )(acts, bias_b)


# speedup vs baseline: 19.6283x; 19.6283x over previous
import functools

import jax
import jax.numpy as jnp
from jax import lax
from jax.experimental import pallas as pl
from jax.experimental.pallas import tpu as pltpu
from jax.experimental.pallas import tpu_sc as plsc

BATCH = 1024
INPUTS = 4096
CLASSES = 10
FIN = 16
ENTRIES = 8192
HFN = 2
FPD = INPUTS // FIN

NC, NS, L = 2, 16, 16
NW = NC * NS
ROWS_PER_W = BATCH // NW


def _pack_body(t_ref, o_ref):
    acc = jnp.zeros(o_ref.shape, jnp.int32)
    for c in range(CLASSES):
        acc = acc | jnp.where(t_ref[c] >= 0.0, jnp.int32(1 << c), jnp.int32(0))
    o_ref[...] = acc


def _pack_table(table):
    bf = 32
    return pl.pallas_call(
        _pack_body,
        grid=(FPD // bf,),
        in_specs=[pl.BlockSpec((CLASSES, bf, ENTRIES), lambda i: (0, i, 0))],
        out_specs=pl.BlockSpec((bf, ENTRIES), lambda i: (i, 0)),
        out_shape=jax.ShapeDtypeStruct((FPD, ENTRIES), jnp.int32),
    )(table)


def _sc_body(x_hbm, packed_hbm, gidx_hbm, hvb_hbm, bias_hbm, out_hbm,
             xrow_v, gidx_v, hvb_v, bias_v, idx_v, gath_v, outrow_v, sem):
    wid = lax.axis_index("s") * NC + lax.axis_index("c")

    pltpu.sync_copy(gidx_hbm, gidx_v)
    pltpu.sync_copy(hvb_hbm, hvb_v)
    pltpu.sync_copy(bias_hbm, bias_v)
    bias_vec = bias_v[...]

    lanes = lax.iota(jnp.int32, L)

    def row_body(rr, _):
        r = wid * ROWS_PER_W + rr
        pltpu.sync_copy(x_hbm.at[r], xrow_v)

        hv0 = [hvb_v[pl.ds(i * L, L)] for i in range(FIN)]
        hv1 = [hvb_v[pl.ds(FPD + i * L, L)] for i in range(FIN)]

        for g in range(16):
            acc0 = jnp.zeros((L,), jnp.int32)
            acc1 = jnp.zeros((L,), jnp.int32)
            for i in range(FIN):
                gv = gidx_v[pl.ds(g * 256 + i * L, L)]
                v = plsc.load_gather(xrow_v, [gv])
                acc0 = acc0 ^ (v * hv0[i])
                acc1 = acc1 ^ (v * hv1[i])
            fbase = (lanes + g * L) * ENTRIES
            q = g // 8
            off = (g * L) % 128
            idx_v[q, pl.ds(off, L)] = acc0 + fbase
            idx_v[2 + q, pl.ds(off, L)] = acc1 + fbase

        descs = [pltpu.async_copy(packed_hbm.at[idx_v.at[q]], gath_v.at[q], sem)
                 for q in range(4)]
        for d in descs:
            d.wait()

        accv = [jnp.zeros((L,), jnp.int32) for _ in range(CLASSES)]
        for j in range(16):
            q = j // 8
            off = (j * L) % 128
            w0 = gath_v[q, pl.ds(off, L)]
            w1 = gath_v[2 + q, pl.ds(off, L)]
            m = w0 & w1
            for c in range(CLASSES):
                accv[c] = accv[c] + ((m >> c) & 1)

        cnts = jnp.zeros((L,), jnp.int32)
        for c in range(CLASSES):
            cnt = jnp.sum(accv[c])
            cnts = jnp.where(lanes == c, cnt, cnts)
        outrow_v[...] = (2.0 * cnts.astype(jnp.float32)
                         - jnp.float32(FPD) + bias_vec)
        pltpu.sync_copy(outrow_v, out_hbm.at[r])
        return ()

    lax.fori_loop(0, ROWS_PER_W, row_body, ())


@functools.partial(jax.jit, static_argnames=())
def _sc_run(x_b, packed_flat, gidx, hvb, bias_pad):
    mesh = plsc.VectorSubcoreMesh(core_axis_name="c", subcore_axis_name="s")
    f = pl.kernel(
        _sc_body,
        out_type=jax.ShapeDtypeStruct((BATCH, L), jnp.float32),
        mesh=mesh,
        compiler_params=pltpu.CompilerParams(needs_layout_passes=False),
        scratch_types=[
            pltpu.VMEM((INPUTS,), jnp.int32),
            pltpu.VMEM((INPUTS,), jnp.int32),
            pltpu.VMEM((HFN * FPD,), jnp.int32),
            pltpu.VMEM((L,), jnp.float32),
            pltpu.VMEM((4, 128), jnp.int32),
            pltpu.VMEM((4, 128), jnp.int32),
            pltpu.VMEM((L,), jnp.float32),
            pltpu.SemaphoreType.DMA,
        ],
    )
    return f(x_b, packed_flat, gidx, hvb, bias_pad)


def kernel(x_b, table, hash_values, input_order, bias):
    packed = _pack_table(table)
    packed_flat = packed.reshape(-1)
    gidx = input_order.reshape(16, 16, 16).transpose(0, 2, 1).reshape(-1)
    hvb = jnp.broadcast_to(hash_values[:, :, None], (HFN, FIN, L)).reshape(-1)
    bias_pad = jnp.pad(bias, (0, L - CLASSES))
    out = _sc_run(x_b, packed_flat, gidx, hvb, bias_pad)
    return out[:, :CLASSES]


# speedup vs baseline: 24.0955x; 1.2276x over previous
import functools

import jax
import jax.numpy as jnp
from jax import lax
from jax.experimental import pallas as pl
from jax.experimental.pallas import tpu as pltpu
from jax.experimental.pallas import tpu_sc as plsc

BATCH = 1024
INPUTS = 4096
CLASSES = 10
FIN = 16
ENTRIES = 8192
HFN = 2
FPD = INPUTS // FIN

NC, NS, L = 2, 16, 16
NW = NC * NS
ROWS_PER_W = BATCH // NW


def _pack_body(t_ref, o_ref):
    acc = jnp.zeros(o_ref.shape, jnp.int32)
    for c in range(CLASSES):
        acc = acc | jnp.where(t_ref[c] >= 0.0, jnp.int32(1 << c), jnp.int32(0))
    o_ref[...] = acc


def _pack_table(table):
    bf = 32
    return pl.pallas_call(
        _pack_body,
        grid=(FPD // bf,),
        in_specs=[pl.BlockSpec((CLASSES, bf, ENTRIES), lambda i: (0, i, 0))],
        out_specs=pl.BlockSpec((bf, ENTRIES), lambda i: (i, 0)),
        out_shape=jax.ShapeDtypeStruct((FPD, ENTRIES), jnp.int32),
    )(table)


def _sc_body(x_hbm, packed_hbm, gidx_hbm, hvb_hbm, bias_hbm, out_hbm,
             xrow0_v, xrow1_v, gidx_v, hvb_v, bias_v,
             idx0_v, idx1_v, gath0_v, gath1_v, outbuf_v,
             sem_x0, sem_x1, sem_g0, sem_g1):
    wid = lax.axis_index("s") * NC + lax.axis_index("c")
    base = wid * ROWS_PER_W

    pltpu.sync_copy(gidx_hbm, gidx_v)
    pltpu.sync_copy(hvb_hbm, hvb_v)
    pltpu.sync_copy(bias_hbm, bias_v)
    bias_vec = bias_v[...]
    lanes = lax.iota(jnp.int32, L)

    hv0 = [hvb_v[pl.ds(i * L, L)] for i in range(FIN)]
    hv1 = [hvb_v[pl.ds(FPD + i * L, L)] for i in range(FIN)]

    def hash_row(xrow_v, idx_v):
        for g in range(16):
            acc0 = jnp.zeros((L,), jnp.int32)
            acc1 = jnp.zeros((L,), jnp.int32)
            for i in range(FIN):
                gv = gidx_v[pl.ds(g * 256 + i * L, L)]
                v = plsc.load_gather(xrow_v, [gv])
                acc0 = acc0 ^ (v * hv0[i])
                acc1 = acc1 ^ (v * hv1[i])
            fbase = (lanes + g * L) * ENTRIES
            q = g // 8
            off = (g * L) % 128
            idx_v[q, pl.ds(off, L)] = acc0 + fbase
            idx_v[2 + q, pl.ds(off, L)] = acc1 + fbase

    def fire_g(idx_v, gath_v, sem):
        return [pltpu.async_copy(packed_hbm.at[idx_v.at[q]], gath_v.at[q], sem)
                for q in range(4)]

    def combine(gath_v, rel):
        accv = [jnp.zeros((L,), jnp.int32) for _ in range(CLASSES)]
        for j in range(16):
            q = j // 8
            off = (j * L) % 128
            w0 = gath_v[q, pl.ds(off, L)]
            w1 = gath_v[2 + q, pl.ds(off, L)]
            m = w0 & w1
            for c in range(CLASSES):
                accv[c] = accv[c] + ((m >> c) & 1)
        cnts = jnp.zeros((L,), jnp.int32)
        for c in range(CLASSES):
            cnt = jnp.sum(accv[c])
            cnts = jnp.where(lanes == c, cnt, cnts)
        outbuf_v[pl.ds(rel * L, L)] = (2.0 * cnts.astype(jnp.float32)
                                       - jnp.float32(FPD) + bias_vec)

    pltpu.async_copy(x_hbm.at[base], xrow0_v, sem_x0)

    def body(t, _):
        a = base + 2 * t
        pltpu.make_async_copy(x_hbm.at[a], xrow0_v, sem_x0).wait()
        pltpu.async_copy(x_hbm.at[a + 1], xrow1_v, sem_x1)
        hash_row(xrow0_v, idx0_v)
        dA = fire_g(idx0_v, gath0_v, sem_g0)
        pltpu.make_async_copy(x_hbm.at[a + 1], xrow1_v, sem_x1).wait()

        @pl.when(t < ROWS_PER_W // 2 - 1)
        def _():
            pltpu.async_copy(x_hbm.at[a + 2], xrow0_v, sem_x0)

        hash_row(xrow1_v, idx1_v)
        dB = fire_g(idx1_v, gath1_v, sem_g1)
        for d in dA:
            d.wait()
        combine(gath0_v, 2 * t)
        for d in dB:
            d.wait()
        combine(gath1_v, 2 * t + 1)
        return ()

    lax.fori_loop(0, ROWS_PER_W // 2, body, ())
    pltpu.sync_copy(outbuf_v, out_hbm.at[pl.ds(wid * ROWS_PER_W * L,
                                               ROWS_PER_W * L)])


@functools.partial(jax.jit, static_argnames=())
def _sc_run(x_b, packed_flat, gidx, hvb, bias_pad):
    mesh = plsc.VectorSubcoreMesh(core_axis_name="c", subcore_axis_name="s")
    f = pl.kernel(
        _sc_body,
        out_type=jax.ShapeDtypeStruct((BATCH * L,), jnp.float32),
        mesh=mesh,
        compiler_params=pltpu.CompilerParams(needs_layout_passes=False),
        scratch_types=[
            pltpu.VMEM((INPUTS,), jnp.int32),
            pltpu.VMEM((INPUTS,), jnp.int32),
            pltpu.VMEM((INPUTS,), jnp.int32),
            pltpu.VMEM((HFN * FPD,), jnp.int32),
            pltpu.VMEM((L,), jnp.float32),
            pltpu.VMEM((4, 128), jnp.int32),
            pltpu.VMEM((4, 128), jnp.int32),
            pltpu.VMEM((4, 128), jnp.int32),
            pltpu.VMEM((4, 128), jnp.int32),
            pltpu.VMEM((ROWS_PER_W * L,), jnp.float32),
            pltpu.SemaphoreType.DMA,
            pltpu.SemaphoreType.DMA,
            pltpu.SemaphoreType.DMA,
            pltpu.SemaphoreType.DMA,
        ],
    )
    return f(x_b, packed_flat, gidx, hvb, bias_pad)


def kernel(x_b, table, hash_values, input_order, bias):
    packed = _pack_table(table)
    packed_flat = packed.reshape(-1)
    gidx = input_order.reshape(16, 16, 16).transpose(0, 2, 1).reshape(-1)
    hvb = jnp.broadcast_to(hash_values[:, :, None], (HFN, FIN, L)).reshape(-1)
    bias_pad = jnp.pad(bias, (0, L - CLASSES))
    out = _sc_run(x_b, packed_flat, gidx, hvb, bias_pad)
    return out.reshape(BATCH, L)[:, :CLASSES]
